# Initial kernel scaffold; baseline (speedup 1.0000x reference)
import jax
import jax.numpy as jnp
from jax.experimental import pallas as pl


def kernel(log_probs, targets, input_lengths, target_lengths, forget_rate):
    raise NotImplementedError("write your pallas kernel here")


# speedup vs baseline: 15.4947x; 15.4947x over previous
import functools

import jax
import jax.numpy as jnp
from jax.experimental import pallas as pl
from jax.experimental.pallas import tpu as pltpu

NEG = -1e30


def _ctc_fwd_kernel(lp_ref, tp_ref, mc_ref, me_ref, me1_ref, out_ref,
                    alpha_ref, adot_ref, *, tc, sp, bc, v):
    it = pl.program_id(1)
    nt = pl.num_programs(1)

    tp = tp_ref[...]
    mc_f = mc_ref[...]
    lane = jax.lax.broadcasted_iota(jnp.int32, (bc, sp), 1)
    mb = lane >= 1
    mc = mc_f != 0.0

    def emit_at(t):
        row = lp_ref[t]
        parts = [
            jnp.take_along_axis(row, tp[:, j * v:(j + 1) * v], axis=1)
            for j in range(sp // v)
        ]
        return jnp.concatenate(parts, axis=1)

    @pl.when(it == 0)
    def _init():
        e0 = emit_at(0)
        alpha_ref[...] = jnp.where(lane < 2, e0, NEG)
        adot_ref[...] = jnp.where(lane < 2, e0, 0.0)

    t_start = jnp.where(it == 0, 1, 0)

    def step(t, carry):
        al, ad = carry
        e = emit_at(t)
        b1 = jnp.where(mb, pltpu.roll(al, 1, axis=1), NEG)
        c2 = jnp.where(mc, pltpu.roll(al, 2, axis=1), NEG)
        d1 = pltpu.roll(ad, 1, axis=1)
        d2 = pltpu.roll(ad, 2, axis=1)
        m = jnp.maximum(jnp.maximum(al, b1), c2)
        ea = jnp.exp(al - m)
        eb = jnp.exp(b1 - m)
        ec = jnp.exp(c2 - m)
        den = ea + eb + ec
        num = ea * ad + eb * d1 + ec * d2
        new_al = m + jnp.log(den) + e
        new_ad = num / den + e
        return new_al, new_ad

    carry0 = (alpha_ref[...], adot_ref[...])
    a_fin, d_fin = jax.lax.fori_loop(t_start, tc, step, carry0)
    alpha_ref[...] = a_fin
    adot_ref[...] = d_fin

    @pl.when(it == nt - 1)
    def _readout():
        me = me_ref[...]
        me1 = me1_ref[...]
        a = jnp.sum(a_fin * me, axis=1, keepdims=True)
        b = jnp.sum(a_fin * me1, axis=1, keepdims=True)
        da = jnp.sum(d_fin * me, axis=1, keepdims=True)
        db = jnp.sum(d_fin * me1, axis=1, keepdims=True)
        mm = jnp.maximum(a, b)
        wa = jnp.exp(a - mm)
        wb = jnp.exp(b - mm)
        contrib = (wa * da + wb * db) / (wa + wb)
        out_ref[...] = jnp.broadcast_to(contrib[None], (1, bc, 128))


def kernel(log_probs, targets, input_lengths, target_lengths, forget_rate):
    t_dim, b_dim, v_dim = log_probs.shape
    l_dim = targets.shape[1]
    s_dim = 2 * l_dim + 1
    sp = ((s_dim + 127) // 128) * 128

    bc = 16 if b_dim % 16 == 0 else b_dim
    tc = 256 if t_dim % 256 == 0 else t_dim

    tgt = targets.astype(jnp.int32)
    tl = target_lengths.astype(jnp.int32)

    s_iota = jnp.arange(sp, dtype=jnp.int32)[None, :]
    lab_idx = jnp.clip((s_iota - 1) // 2, 0, l_dim - 1)
    gathered = jnp.take_along_axis(tgt, jnp.broadcast_to(lab_idx, (b_dim, sp)), axis=1)
    odd = (s_iota % 2 == 1) & (s_iota < s_dim)
    tp = jnp.where(odd, gathered, 0)
    tp_m2 = jnp.pad(tp, ((0, 0), (2, 0)))[:, :sp]
    mc = ((s_iota >= 2) & (tp != 0) & (tp != tp_m2)).astype(jnp.float32)

    s_end = 2 * tl[:, None]
    me = (s_iota == s_end).astype(jnp.float32)
    me1 = (s_iota == s_end - 1).astype(jnp.float32)

    grid = (b_dim // bc, t_dim // tc)
    out = pl.pallas_call(
        functools.partial(_ctc_fwd_kernel, tc=tc, sp=sp, bc=bc, v=v_dim),
        grid=grid,
        in_specs=[
            pl.BlockSpec((tc, bc, v_dim), lambda ib, it: (it, ib, 0)),
            pl.BlockSpec((bc, sp), lambda ib, it: (ib, 0)),
            pl.BlockSpec((bc, sp), lambda ib, it: (ib, 0)),
            pl.BlockSpec((bc, sp), lambda ib, it: (ib, 0)),
            pl.BlockSpec((bc, sp), lambda ib, it: (ib, 0)),
        ],
        out_specs=pl.BlockSpec((1, bc, 128), lambda ib, it: (ib, 0, 0)),
        out_shape=jax.ShapeDtypeStruct((b_dim // bc, bc, 128), jnp.float32),
        scratch_shapes=[
            pltpu.VMEM((bc, sp), jnp.float32),
            pltpu.VMEM((bc, sp), jnp.float32),
        ],
        compiler_params=pltpu.CompilerParams(
            dimension_semantics=("parallel", "arbitrary"),
        ),
    )(log_probs, tp, mc, me, me1)

    return out[:, :, 0].sum() / b_dim / -4
